# baseline (speedup 1.0000x reference)
import functools
import jax
import jax.numpy as jnp
from jax import lax
from jax.experimental import pallas as pl
from jax.experimental.pallas import tpu as pltpu
from jax.experimental.pallas import tpu_sc as plsc

N_NODES = 10000
N_EDGES = 320000
H = 128
E_DIM = 16
OUT_DIM = 2 * H

NC = 2
NS = 16
NW = NC * NS
EDGES_PER_TILE = N_EDGES // NW
CHUNK = 80
NCHUNK = EDGES_PER_TILE // CHUNK

BLK_N = 1000


def _node_mm_body(nf_ref, w_ref, p1_ref, p2_ref):
    nf = nf_ref[...]
    p1_ref[...] = jnp.dot(nf, w_ref[:, :OUT_DIM],
                          preferred_element_type=jnp.float32).astype(jnp.bfloat16)
    p2_ref[...] = jnp.dot(nf, w_ref[:, OUT_DIM:],
                          preferred_element_type=jnp.float32).astype(jnp.bfloat16)


def _node_matmul(nf, w2):
    grid = N_NODES // BLK_N
    return pl.pallas_call(
        _node_mm_body,
        grid=(grid,),
        in_specs=[
            pl.BlockSpec((BLK_N, H), lambda i: (i, 0)),
            pl.BlockSpec((H, 2 * OUT_DIM), lambda i: (0, 0)),
        ],
        out_specs=[
            pl.BlockSpec((BLK_N, OUT_DIM), lambda i: (i, 0)),
            pl.BlockSpec((BLK_N, OUT_DIM), lambda i: (i, 0)),
        ],
        out_shape=[
            jax.ShapeDtypeStruct((N_NODES, OUT_DIM), jnp.bfloat16),
            jax.ShapeDtypeStruct((N_NODES, OUT_DIM), jnp.bfloat16),
        ],
    )(nf, w2)


def _gather_body(p1_hbm, p2_hbm, src_hbm, dst_hbm, zcat_hbm,
                 sidx, didx, zbuf, gbuf, sem):
    cid = lax.axis_index("c")
    sid = lax.axis_index("s")
    wid = sid * NC + cid
    base = wid * EDGES_PER_TILE

    pltpu.sync_copy(src_hbm.at[wid], sidx)
    pltpu.sync_copy(dst_hbm.at[wid], didx)

    def fire(j, s):
        pltpu.async_copy(p1_hbm.at[sidx.at[j]], zbuf.at[s], sem.at[s])
        pltpu.async_copy(p2_hbm.at[didx.at[j]], gbuf.at[s], sem.at[s])

    def drain(j, s):
        pltpu.make_async_copy(p1_hbm.at[sidx.at[j]], zbuf.at[s], sem.at[s]).wait()
        pltpu.make_async_copy(p2_hbm.at[didx.at[j]], gbuf.at[s], sem.at[s]).wait()

    def process(j, s):
        drain(j, s)
        rows = pl.ds(base + j * CHUNK, CHUNK)
        pltpu.sync_copy(zbuf.at[s], zcat_hbm.at[0, rows])
        pltpu.sync_copy(gbuf.at[s], zcat_hbm.at[1, rows])

    fire(0, 0)
    fire(1, 1)

    def step(j2, carry):
        for b in range(2):
            j = 2 * j2 + b
            process(j, b)

            @pl.when(j < NCHUNK - 2)
            def _():
                fire(j + 2, b)
        return carry

    lax.fori_loop(0, (NCHUNK - 1) // 2, step, 0)
    process(NCHUNK - 1, (NCHUNK - 1) % 2)


def _sc_gather(p1, p2, src3d, dst3d):
    mesh = plsc.VectorSubcoreMesh(core_axis_name="c", subcore_axis_name="s")
    f = pl.kernel(
        _gather_body,
        out_type=jax.ShapeDtypeStruct((2, N_EDGES, H), jnp.int32),
        mesh=mesh,
        scratch_types=[
            pltpu.VMEM((NCHUNK, CHUNK), jnp.int32),
            pltpu.VMEM((NCHUNK, CHUNK), jnp.int32),
            pltpu.VMEM((2, CHUNK, H), jnp.int32),
            pltpu.VMEM((2, CHUNK, H), jnp.int32),
            pltpu.SemaphoreType.DMA((2,)),
        ],
    )
    return f(p1, p2, src3d, dst3d)


BLK_E = 2000
E_GRID = N_EDGES // BLK_E


def _edge_term(g1, g2, ef, w3, bvec):
    return (g1[0].astype(jnp.float32) + g2[0].astype(jnp.float32)
            + jnp.dot(ef, w3, preferred_element_type=jnp.float32) + bvec)


def _stats_body(g1_ref, g2_ref, ef_ref, w3_ref, b_ref, out_ref, acc):
    i = pl.program_id(0)

    @pl.when(i == 0)
    def _():
        acc[...] = jnp.zeros_like(acc)

    z = _edge_term(g1_ref[...], g2_ref[...], ef_ref[...], w3_ref[...],
                   b_ref[...])
    acc[0:1, :] += jnp.sum(z, axis=0, keepdims=True)
    acc[1:2, :] += jnp.sum(z * z, axis=0, keepdims=True)

    @pl.when(i == E_GRID - 1)
    def _():
        out_ref[...] = acc[...]


def _stats(zcat, ef, w3, bvec):
    return pl.pallas_call(
        _stats_body,
        grid=(E_GRID,),
        in_specs=[
            pl.BlockSpec((1, BLK_E, OUT_DIM), lambda i: (0, i, 0)),
            pl.BlockSpec((1, BLK_E, OUT_DIM), lambda i: (1, i, 0)),
            pl.BlockSpec((BLK_E, E_DIM), lambda i: (i, 0)),
            pl.BlockSpec((E_DIM, OUT_DIM), lambda i: (0, 0)),
            pl.BlockSpec((1, OUT_DIM), lambda i: (0, 0)),
        ],
        out_specs=pl.BlockSpec((8, OUT_DIM), lambda i: (0, 0)),
        out_shape=jax.ShapeDtypeStruct((8, OUT_DIM), jnp.float32),
        scratch_shapes=[pltpu.VMEM((8, OUT_DIM), jnp.float32)],
    )(zcat, zcat, ef, w3, bvec)


def _msg_body(g1_ref, g2_ref, ef_ref, w3_ref, b_ref, stats_ref, gam_ref,
              bet_ref, msg_ref):
    z = _edge_term(g1_ref[...], g2_ref[...], ef_ref[...], w3_ref[...],
                   b_ref[...])
    mean = stats_ref[0:1, :] * (1.0 / N_EDGES)
    var = stats_ref[1:2, :] * (1.0 / N_EDGES) - mean * mean
    scale = gam_ref[...] * lax.rsqrt(var + 1e-5)
    shift = bet_ref[...] - mean * scale
    zn = z * scale + shift
    sig = jax.nn.sigmoid(zn[:, :H])
    xp = zn[:, H:]
    sp = jnp.maximum(xp, 0.0) + jnp.log1p(jnp.exp(-jnp.abs(xp)))
    msg_ref[...] = sig * sp


def _msg(zcat, ef, w3, bvec, stats, gamma, beta):
    return pl.pallas_call(
        _msg_body,
        grid=(E_GRID,),
        in_specs=[
            pl.BlockSpec((1, BLK_E, OUT_DIM), lambda i: (0, i, 0)),
            pl.BlockSpec((1, BLK_E, OUT_DIM), lambda i: (1, i, 0)),
            pl.BlockSpec((BLK_E, E_DIM), lambda i: (i, 0)),
            pl.BlockSpec((E_DIM, OUT_DIM), lambda i: (0, 0)),
            pl.BlockSpec((1, OUT_DIM), lambda i: (0, 0)),
            pl.BlockSpec((8, OUT_DIM), lambda i: (0, 0)),
            pl.BlockSpec((1, OUT_DIM), lambda i: (0, 0)),
            pl.BlockSpec((1, OUT_DIM), lambda i: (0, 0)),
        ],
        out_specs=pl.BlockSpec((BLK_E, H), lambda i: (i, 0)),
        out_shape=jax.ShapeDtypeStruct((N_EDGES, H), jnp.float32),
    )(zcat, zcat, ef, w3, bvec, stats, gamma, beta)


N_NODES_PAD = 10240
ROWS_PER_TILE = N_NODES_PAD // NS


def _scatter_body(msg_hbm, src_hbm, zeros_hbm, parts_hbm, sidx, mbuf, acc, sem):
    cid = lax.axis_index("c")
    sid = lax.axis_index("s")
    wid = sid * NC + cid
    base = wid * EDGES_PER_TILE
    stripe = sid * ROWS_PER_TILE

    pltpu.sync_copy(src_hbm.at[wid], sidx)
    pltpu.sync_copy(zeros_hbm.at[pl.ds(stripe, ROWS_PER_TILE)],
                    acc.at[pl.ds(stripe, ROWS_PER_TILE)])
    plsc.subcore_barrier()

    def fire(j, s):
        pltpu.async_copy(msg_hbm.at[pl.ds(base + j * CHUNK, CHUNK)],
                         mbuf.at[s], sem.at[s])

    def process(j, s):
        pltpu.make_async_copy(msg_hbm.at[pl.ds(base + j * CHUNK, CHUNK)],
                              mbuf.at[s], sem.at[s]).wait()
        pltpu.sync_copy(mbuf.at[s], acc.at[sidx.at[j]], add=True)

    fire(0, 0)
    fire(1, 1)

    def chunk(j2, carry):
        for b in range(2):
            j = 2 * j2 + b
            process(j, b)

            @pl.when(j < NCHUNK - 2)
            def _():
                fire(j + 2, b)
        return carry

    lax.fori_loop(0, (NCHUNK - 1) // 2, chunk, 0)
    process(NCHUNK - 1, (NCHUNK - 1) % 2)
    plsc.subcore_barrier()
    pltpu.sync_copy(acc.at[pl.ds(stripe, ROWS_PER_TILE)],
                    parts_hbm.at[cid, pl.ds(stripe, ROWS_PER_TILE)])


def _sc_scatter(msg, src3d, zeros):
    mesh = plsc.VectorSubcoreMesh(core_axis_name="c", subcore_axis_name="s")
    f = pl.kernel(
        _scatter_body,
        out_type=jax.ShapeDtypeStruct((NC, N_NODES_PAD, H), jnp.float32),
        mesh=mesh,
        scratch_types=[
            pltpu.VMEM((NCHUNK, CHUNK), jnp.int32),
            pltpu.VMEM((2, CHUNK, H), jnp.float32),
            pltpu.VMEM_SHARED((N_NODES_PAD, H), jnp.float32),
            pltpu.SemaphoreType.DMA((2,)),
        ],
    )
    return f(msg, src3d, zeros)


def _final_body(nf_ref, p0_ref, p1_ref, out_ref):
    out_ref[...] = nf_ref[...] + p0_ref[0] + p1_ref[0]


def _final_add(nf, parts):
    grid = N_NODES // BLK_N
    return pl.pallas_call(
        _final_body,
        grid=(grid,),
        in_specs=[
            pl.BlockSpec((BLK_N, H), lambda i: (i, 0)),
            pl.BlockSpec((1, BLK_N, H), lambda i: (0, i, 0)),
            pl.BlockSpec((1, BLK_N, H), lambda i: (1, i, 0)),
        ],
        out_specs=pl.BlockSpec((BLK_N, H), lambda i: (i, 0)),
        out_shape=jax.ShapeDtypeStruct((N_NODES, H), jnp.float32),
    )(nf, parts, parts)


@jax.jit
def kernel(node_feats, edge_index, edge_feats, W, b, gamma, beta):
    src = edge_index[0].astype(jnp.int32)
    dst = edge_index[1].astype(jnp.int32)
    wt = W.T
    w12 = jnp.concatenate([wt[:H], wt[H:2 * H]], axis=1)
    w3 = wt[2 * H:]
    bvec = b.reshape(1, OUT_DIM)
    gam = gamma.reshape(1, OUT_DIM)
    bet = beta.reshape(1, OUT_DIM)
    src3d = src.reshape(NW, NCHUNK, CHUNK)
    dst3d = dst.reshape(NW, NCHUNK, CHUNK)

    p1, p2 = _node_matmul(node_feats, w12)
    p1i = lax.bitcast_convert_type(
        p1.reshape(N_NODES, OUT_DIM // 2, 2), jnp.int32)
    p2i = lax.bitcast_convert_type(
        p2.reshape(N_NODES, OUT_DIM // 2, 2), jnp.int32)
    zcati = _sc_gather(p1i, p2i, src3d, dst3d)
    zcat = lax.bitcast_convert_type(zcati, jnp.bfloat16).reshape(
        2, N_EDGES, OUT_DIM)
    stats = _stats(zcat, edge_feats, w3, bvec)
    msg = _msg(zcat, edge_feats, w3, bvec, stats, gam, bet)
    zeros = jnp.zeros((N_NODES_PAD, H), jnp.float32)
    parts = _sc_scatter(msg, src3d, zeros)
    return _final_add(node_feats, parts)


# speedup vs baseline: 2.7541x; 2.7541x over previous
import jax
import jax.numpy as jnp
from jax import lax
from jax.experimental import pallas as pl
from jax.experimental.pallas import tpu as pltpu
from jax.experimental.pallas import tpu_sc as plsc

N_NODES = 10000
N_EDGES = 320000
H = 128
E_DIM = 16
OUT_DIM = 2 * H

NC = 2
NS = 16
NW = NC * NS
EDGES_PER_TILE = N_EDGES // NW
CHUNK = 80
NCHUNK = EDGES_PER_TILE // CHUNK

_MASK = -65536


def _pack_bf16_pair(lo_f32, hi_f32):
    lob = lax.bitcast_convert_type(lo_f32, jnp.int32) + 32768
    hib = lax.bitcast_convert_type(hi_f32, jnp.int32) + 32768
    return lax.shift_right_logical(lob, 16) | (hib & _MASK)


def _unpack_bf16_pair(packed_i32):
    lo = lax.bitcast_convert_type(packed_i32 << 16, jnp.float32)
    hi = lax.bitcast_convert_type(packed_i32 & _MASK, jnp.float32)
    return lo, hi


BLK_N = 1000


def _node_mm_body(nf_ref, w_ref, p1_ref, p2_ref):
    nf = nf_ref[...]
    a1 = jnp.dot(nf, w_ref[:, :OUT_DIM], preferred_element_type=jnp.float32)
    a2 = jnp.dot(nf, w_ref[:, OUT_DIM:], preferred_element_type=jnp.float32)
    p1_ref[...] = _pack_bf16_pair(a1[:, :H], a1[:, H:])
    p2_ref[...] = _pack_bf16_pair(a2[:, :H], a2[:, H:])


def _node_matmul(nf, w2):
    grid = N_NODES // BLK_N
    return pl.pallas_call(
        _node_mm_body,
        grid=(grid,),
        in_specs=[
            pl.BlockSpec((BLK_N, H), lambda i: (i, 0)),
            pl.BlockSpec((H, 2 * OUT_DIM), lambda i: (0, 0)),
        ],
        out_specs=[
            pl.BlockSpec((BLK_N, H), lambda i: (i, 0)),
            pl.BlockSpec((BLK_N, H), lambda i: (i, 0)),
        ],
        out_shape=[
            jax.ShapeDtypeStruct((N_NODES, H), jnp.int32),
            jax.ShapeDtypeStruct((N_NODES, H), jnp.int32),
        ],
    )(nf, w2)


def _gather_body(p1_hbm, p2_hbm, src_hbm, dst_hbm, zcat_hbm,
                 sidx, didx, zbuf, gbuf, sem):
    cid = lax.axis_index("c")
    sid = lax.axis_index("s")
    wid = sid * NC + cid
    base = wid * EDGES_PER_TILE

    pltpu.sync_copy(src_hbm.at[wid], sidx)
    pltpu.sync_copy(dst_hbm.at[wid], didx)

    def fire(j, s):
        pltpu.async_copy(p1_hbm.at[sidx.at[j]], zbuf.at[s], sem.at[s])
        pltpu.async_copy(p2_hbm.at[didx.at[j]], gbuf.at[s], sem.at[s])

    def drain(j, s):
        pltpu.make_async_copy(p1_hbm.at[sidx.at[j]], zbuf.at[s],
                              sem.at[s]).wait()
        pltpu.make_async_copy(p2_hbm.at[didx.at[j]], gbuf.at[s],
                              sem.at[s]).wait()

    def process(j, s):
        drain(j, s)
        rows = pl.ds(base + j * CHUNK, CHUNK)
        pltpu.sync_copy(zbuf.at[s], zcat_hbm.at[0, rows])
        pltpu.sync_copy(gbuf.at[s], zcat_hbm.at[1, rows])

    fire(0, 0)
    fire(1, 1)

    def step(j2, carry):
        for b in range(2):
            j = 2 * j2 + b
            process(j, b)

            @pl.when(j < NCHUNK - 2)
            def _():
                fire(j + 2, b)
        return carry

    lax.fori_loop(0, (NCHUNK - 1) // 2, step, 0)
    process(NCHUNK - 1, (NCHUNK - 1) % 2)


def _sc_gather(p1, p2, src3d, dst3d):
    mesh = plsc.VectorSubcoreMesh(core_axis_name="c", subcore_axis_name="s")
    f = pl.kernel(
        _gather_body,
        out_type=jax.ShapeDtypeStruct((2, N_EDGES, H), jnp.int32),
        mesh=mesh,
        scratch_types=[
            pltpu.VMEM((NCHUNK, CHUNK), jnp.int32),
            pltpu.VMEM((NCHUNK, CHUNK), jnp.int32),
            pltpu.VMEM((2, CHUNK, H), jnp.int32),
            pltpu.VMEM((2, CHUNK, H), jnp.int32),
            pltpu.SemaphoreType.DMA((2,)),
        ],
    )
    return f(p1, p2, src3d, dst3d)


BLK_E = 2000
E_GRID = N_EDGES // BLK_E


def _z_halves(g1_ref, g2_ref, ef_ref, w3_ref, b_ref):
    a_lo, a_hi = _unpack_bf16_pair(g1_ref[0])
    b_lo, b_hi = _unpack_bf16_pair(g2_ref[0])
    q = jnp.dot(ef_ref[...], w3_ref[...], preferred_element_type=jnp.float32)
    bv = b_ref[...]
    z1 = a_lo + b_lo + q[:, :H] + bv[:, :H]
    z2 = a_hi + b_hi + q[:, H:] + bv[:, H:]
    return z1, z2


def _stats_body(g1_ref, g2_ref, ef_ref, w3_ref, b_ref, out_ref, acc):
    i = pl.program_id(0)

    @pl.when(i == 0)
    def _():
        acc[...] = jnp.zeros_like(acc)

    z1, z2 = _z_halves(g1_ref, g2_ref, ef_ref, w3_ref, b_ref)
    acc[0:1, :H] += jnp.sum(z1, axis=0, keepdims=True)
    acc[0:1, H:] += jnp.sum(z2, axis=0, keepdims=True)
    acc[1:2, :H] += jnp.sum(z1 * z1, axis=0, keepdims=True)
    acc[1:2, H:] += jnp.sum(z2 * z2, axis=0, keepdims=True)

    @pl.when(i == E_GRID - 1)
    def _():
        out_ref[...] = acc[...]


def _stats(zcat, ef, w3, bvec):
    return pl.pallas_call(
        _stats_body,
        grid=(E_GRID,),
        in_specs=[
            pl.BlockSpec((1, BLK_E, H), lambda i: (0, i, 0)),
            pl.BlockSpec((1, BLK_E, H), lambda i: (1, i, 0)),
            pl.BlockSpec((BLK_E, E_DIM), lambda i: (i, 0)),
            pl.BlockSpec((E_DIM, OUT_DIM), lambda i: (0, 0)),
            pl.BlockSpec((1, OUT_DIM), lambda i: (0, 0)),
        ],
        out_specs=pl.BlockSpec((8, OUT_DIM), lambda i: (0, 0)),
        out_shape=jax.ShapeDtypeStruct((8, OUT_DIM), jnp.float32),
        scratch_shapes=[pltpu.VMEM((8, OUT_DIM), jnp.float32)],
    )(zcat, zcat, ef, w3, bvec)


def _msg_body(g1_ref, g2_ref, ef_ref, w3_ref, b_ref, stats_ref, gam_ref,
              bet_ref, msg_ref):
    z1, z2 = _z_halves(g1_ref, g2_ref, ef_ref, w3_ref, b_ref)
    mean = stats_ref[0:1, :] * (1.0 / N_EDGES)
    var = stats_ref[1:2, :] * (1.0 / N_EDGES) - mean * mean
    scale = gam_ref[...] * lax.rsqrt(var + 1e-5)
    shift = bet_ref[...] - mean * scale
    zn1 = z1 * scale[:, :H] + shift[:, :H]
    zn2 = z2 * scale[:, H:] + shift[:, H:]
    sig = jax.nn.sigmoid(zn1)
    sp = jnp.maximum(zn2, 0.0) + jnp.log1p(jnp.exp(-jnp.abs(zn2)))
    msg_ref[...] = sig * sp


def _msg(zcat, ef, w3, bvec, stats, gamma, beta):
    return pl.pallas_call(
        _msg_body,
        grid=(E_GRID,),
        in_specs=[
            pl.BlockSpec((1, BLK_E, H), lambda i: (0, i, 0)),
            pl.BlockSpec((1, BLK_E, H), lambda i: (1, i, 0)),
            pl.BlockSpec((BLK_E, E_DIM), lambda i: (i, 0)),
            pl.BlockSpec((E_DIM, OUT_DIM), lambda i: (0, 0)),
            pl.BlockSpec((1, OUT_DIM), lambda i: (0, 0)),
            pl.BlockSpec((8, OUT_DIM), lambda i: (0, 0)),
            pl.BlockSpec((1, OUT_DIM), lambda i: (0, 0)),
            pl.BlockSpec((1, OUT_DIM), lambda i: (0, 0)),
        ],
        out_specs=pl.BlockSpec((BLK_E, H), lambda i: (i, 0)),
        out_shape=jax.ShapeDtypeStruct((N_EDGES, H), jnp.float32),
    )(zcat, zcat, ef, w3, bvec, stats, gamma, beta)


N_NODES_PAD = 10240
ROWS_PER_TILE = N_NODES_PAD // NS


def _scatter_body(msg_hbm, src_hbm, zeros_hbm, parts_hbm, sidx, mbuf, acc,
                  sem):
    cid = lax.axis_index("c")
    sid = lax.axis_index("s")
    wid = sid * NC + cid
    base = wid * EDGES_PER_TILE
    stripe = sid * ROWS_PER_TILE

    pltpu.sync_copy(src_hbm.at[wid], sidx)
    pltpu.sync_copy(zeros_hbm.at[pl.ds(stripe, ROWS_PER_TILE)],
                    acc.at[pl.ds(stripe, ROWS_PER_TILE)])
    plsc.subcore_barrier()

    def fire(j, s):
        pltpu.async_copy(msg_hbm.at[pl.ds(base + j * CHUNK, CHUNK)],
                         mbuf.at[s], sem.at[s])

    def process(j, s):
        pltpu.make_async_copy(msg_hbm.at[pl.ds(base + j * CHUNK, CHUNK)],
                              mbuf.at[s], sem.at[s]).wait()
        pltpu.sync_copy(mbuf.at[s], acc.at[sidx.at[j]], add=True)

    fire(0, 0)
    fire(1, 1)

    def chunk(j2, carry):
        for b in range(2):
            j = 2 * j2 + b
            process(j, b)

            @pl.when(j < NCHUNK - 2)
            def _():
                fire(j + 2, b)
        return carry

    lax.fori_loop(0, (NCHUNK - 1) // 2, chunk, 0)
    process(NCHUNK - 1, (NCHUNK - 1) % 2)
    plsc.subcore_barrier()
    pltpu.sync_copy(acc.at[pl.ds(stripe, ROWS_PER_TILE)],
                    parts_hbm.at[cid, pl.ds(stripe, ROWS_PER_TILE)])


def _sc_scatter(msg, src3d, zeros):
    mesh = plsc.VectorSubcoreMesh(core_axis_name="c", subcore_axis_name="s")
    f = pl.kernel(
        _scatter_body,
        out_type=jax.ShapeDtypeStruct((NC, N_NODES_PAD, H), jnp.float32),
        mesh=mesh,
        scratch_types=[
            pltpu.VMEM((NCHUNK, CHUNK), jnp.int32),
            pltpu.VMEM((2, CHUNK, H), jnp.float32),
            pltpu.VMEM_SHARED((N_NODES_PAD, H), jnp.float32),
            pltpu.SemaphoreType.DMA((2,)),
        ],
    )
    return f(msg, src3d, zeros)


def _final_body(nf_ref, p0_ref, p1_ref, out_ref):
    out_ref[...] = nf_ref[...] + p0_ref[0] + p1_ref[0]


def _final_add(nf, parts):
    grid = N_NODES // BLK_N
    return pl.pallas_call(
        _final_body,
        grid=(grid,),
        in_specs=[
            pl.BlockSpec((BLK_N, H), lambda i: (i, 0)),
            pl.BlockSpec((1, BLK_N, H), lambda i: (0, i, 0)),
            pl.BlockSpec((1, BLK_N, H), lambda i: (1, i, 0)),
        ],
        out_specs=pl.BlockSpec((BLK_N, H), lambda i: (i, 0)),
        out_shape=jax.ShapeDtypeStruct((N_NODES, H), jnp.float32),
    )(nf, parts, parts)


@jax.jit
def kernel(node_feats, edge_index, edge_feats, W, b, gamma, beta):
    src = edge_index[0].astype(jnp.int32)
    dst = edge_index[1].astype(jnp.int32)
    wt = W.T
    w12 = jnp.concatenate([wt[:H], wt[H:2 * H]], axis=1)
    w3 = wt[2 * H:]
    bvec = b.reshape(1, OUT_DIM)
    gam = gamma.reshape(1, OUT_DIM)
    bet = beta.reshape(1, OUT_DIM)
    src3d = src.reshape(NW, NCHUNK, CHUNK)
    dst3d = dst.reshape(NW, NCHUNK, CHUNK)

    p1, p2 = _node_matmul(node_feats, w12)
    zcat = _sc_gather(p1, p2, src3d, dst3d)
    stats = _stats(zcat, edge_feats, w3, bvec)
    msg = _msg(zcat, edge_feats, w3, bvec, stats, gam, bet)
    zeros = jnp.zeros((N_NODES_PAD, H), jnp.float32)
    parts = _sc_scatter(msg, src3d, zeros)
    return _final_add(node_feats, parts)


# speedup vs baseline: 2.7698x; 1.0057x over previous
import jax
import jax.numpy as jnp
from jax import lax
from jax.experimental import pallas as pl
from jax.experimental.pallas import tpu as pltpu
from jax.experimental.pallas import tpu_sc as plsc

N_NODES = 10000
N_EDGES = 320000
H = 128
E_DIM = 16
OUT_DIM = 2 * H

NC = 2
NS = 16
NW = NC * NS
EDGES_PER_TILE = N_EDGES // NW
CHUNK = 80
NCHUNK = EDGES_PER_TILE // CHUNK

_MASK = -65536


def _pack_bf16_pair(lo_f32, hi_f32):
    lob = lax.bitcast_convert_type(lo_f32, jnp.int32) + 32768
    hib = lax.bitcast_convert_type(hi_f32, jnp.int32) + 32768
    return lax.shift_right_logical(lob, 16) | (hib & _MASK)


def _unpack_bf16_pair(packed_i32):
    lo = lax.bitcast_convert_type(packed_i32 << 16, jnp.float32)
    hi = lax.bitcast_convert_type(packed_i32 & _MASK, jnp.float32)
    return lo, hi


BLK_N = 1000


def _node_mm_body(nf_ref, w_ref, p1_ref, p2_ref):
    nf = nf_ref[...]
    dn = (((1,), (1,)), ((), ()))
    a1 = lax.dot_general(nf, w_ref[:, :H], dn,
                         preferred_element_type=jnp.float32)
    a2 = lax.dot_general(nf, w_ref[:, H:2 * H], dn,
                         preferred_element_type=jnp.float32)
    p1_ref[...] = _pack_bf16_pair(a1[:, :H], a1[:, H:])
    p2_ref[...] = _pack_bf16_pair(a2[:, :H], a2[:, H:])


def _node_matmul(nf, w):
    grid = N_NODES // BLK_N
    return pl.pallas_call(
        _node_mm_body,
        grid=(grid,),
        in_specs=[
            pl.BlockSpec((BLK_N, H), lambda i: (i, 0)),
            pl.BlockSpec((OUT_DIM, 2 * H + E_DIM), lambda i: (0, 0)),
        ],
        out_specs=[
            pl.BlockSpec((BLK_N, H), lambda i: (i, 0)),
            pl.BlockSpec((BLK_N, H), lambda i: (i, 0)),
        ],
        out_shape=[
            jax.ShapeDtypeStruct((N_NODES, H), jnp.int32),
            jax.ShapeDtypeStruct((N_NODES, H), jnp.int32),
        ],
    )(nf, w)


def _gather_body(p1_hbm, p2_hbm, eidx_hbm, zcat_hbm,
                 sidx, didx, zbuf, gbuf, sem):
    cid = lax.axis_index("c")
    sid = lax.axis_index("s")
    wid = sid * NC + cid
    base = wid * EDGES_PER_TILE

    pltpu.sync_copy(eidx_hbm.at[0, wid], sidx)
    pltpu.sync_copy(eidx_hbm.at[1, wid], didx)

    def fire(j, s):
        pltpu.async_copy(p1_hbm.at[sidx.at[j]], zbuf.at[s], sem.at[s])
        pltpu.async_copy(p2_hbm.at[didx.at[j]], gbuf.at[s], sem.at[s])

    def drain(j, s):
        pltpu.make_async_copy(p1_hbm.at[sidx.at[j]], zbuf.at[s],
                              sem.at[s]).wait()
        pltpu.make_async_copy(p2_hbm.at[didx.at[j]], gbuf.at[s],
                              sem.at[s]).wait()

    def process(j, s):
        drain(j, s)
        rows = pl.ds(base + j * CHUNK, CHUNK)
        pltpu.sync_copy(zbuf.at[s], zcat_hbm.at[0, rows])
        pltpu.sync_copy(gbuf.at[s], zcat_hbm.at[1, rows])

    fire(0, 0)
    fire(1, 1)

    def step(j2, carry):
        for b in range(2):
            j = 2 * j2 + b
            process(j, b)

            @pl.when(j < NCHUNK - 2)
            def _():
                fire(j + 2, b)
        return carry

    lax.fori_loop(0, (NCHUNK - 1) // 2, step, 0)
    process(NCHUNK - 1, (NCHUNK - 1) % 2)


def _sc_gather(p1, p2, eidx4):
    mesh = plsc.VectorSubcoreMesh(core_axis_name="c", subcore_axis_name="s")
    f = pl.kernel(
        _gather_body,
        out_type=jax.ShapeDtypeStruct((2, N_EDGES, H), jnp.int32),
        mesh=mesh,
        scratch_types=[
            pltpu.VMEM((NCHUNK, CHUNK), jnp.int32),
            pltpu.VMEM((NCHUNK, CHUNK), jnp.int32),
            pltpu.VMEM((2, CHUNK, H), jnp.int32),
            pltpu.VMEM((2, CHUNK, H), jnp.int32),
            pltpu.SemaphoreType.DMA((2,)),
        ],
    )
    return f(p1, p2, eidx4)


BLK_E = 2000
E_GRID = N_EDGES // BLK_E


def _z_halves(g1_ref, g2_ref, ef_ref, w_ref, b_ref):
    a_lo, a_hi = _unpack_bf16_pair(g1_ref[0])
    b_lo, b_hi = _unpack_bf16_pair(g2_ref[0])
    q = lax.dot_general(ef_ref[...], w_ref[:, 2 * H:], (((1,), (1,)), ((), ())),
                        preferred_element_type=jnp.float32)
    bv = b_ref[...]
    z1 = a_lo + b_lo + q[:, :H] + bv[:, :H]
    z2 = a_hi + b_hi + q[:, H:] + bv[:, H:]
    return z1, z2


def _stats_body(g1_ref, g2_ref, ef_ref, w_ref, b_ref, out_ref, acc):
    i = pl.program_id(0)

    @pl.when(i == 0)
    def _():
        acc[...] = jnp.zeros_like(acc)

    z1, z2 = _z_halves(g1_ref, g2_ref, ef_ref, w_ref, b_ref)
    acc[0:1, :H] += jnp.sum(z1, axis=0, keepdims=True)
    acc[0:1, H:] += jnp.sum(z2, axis=0, keepdims=True)
    acc[1:2, :H] += jnp.sum(z1 * z1, axis=0, keepdims=True)
    acc[1:2, H:] += jnp.sum(z2 * z2, axis=0, keepdims=True)

    @pl.when(i == E_GRID - 1)
    def _():
        out_ref[...] = acc[...]


def _stats(zcat, ef, w, bvec):
    return pl.pallas_call(
        _stats_body,
        grid=(E_GRID,),
        in_specs=[
            pl.BlockSpec((1, BLK_E, H), lambda i: (0, i, 0)),
            pl.BlockSpec((1, BLK_E, H), lambda i: (1, i, 0)),
            pl.BlockSpec((BLK_E, E_DIM), lambda i: (i, 0)),
            pl.BlockSpec((OUT_DIM, 2 * H + E_DIM), lambda i: (0, 0)),
            pl.BlockSpec((1, OUT_DIM), lambda i: (0, 0)),
        ],
        out_specs=pl.BlockSpec((8, OUT_DIM), lambda i: (0, 0)),
        out_shape=jax.ShapeDtypeStruct((8, OUT_DIM), jnp.float32),
        scratch_shapes=[pltpu.VMEM((8, OUT_DIM), jnp.float32)],
    )(zcat, zcat, ef, w, bvec)


def _msg_body(g1_ref, g2_ref, ef_ref, w_ref, b_ref, stats_ref, gam_ref,
              bet_ref, msg_ref):
    z1, z2 = _z_halves(g1_ref, g2_ref, ef_ref, w_ref, b_ref)
    mean = stats_ref[0:1, :] * (1.0 / N_EDGES)
    var = stats_ref[1:2, :] * (1.0 / N_EDGES) - mean * mean
    scale = gam_ref[...] * lax.rsqrt(var + 1e-5)
    shift = bet_ref[...] - mean * scale
    zn1 = z1 * scale[:, :H] + shift[:, :H]
    zn2 = z2 * scale[:, H:] + shift[:, H:]
    sig = jax.nn.sigmoid(zn1)
    sp = jnp.maximum(zn2, 0.0) + jnp.log1p(jnp.exp(-jnp.abs(zn2)))
    msg_ref[...] = sig * sp


def _msg(zcat, ef, w, bvec, stats, gamma, beta):
    return pl.pallas_call(
        _msg_body,
        grid=(E_GRID,),
        in_specs=[
            pl.BlockSpec((1, BLK_E, H), lambda i: (0, i, 0)),
            pl.BlockSpec((1, BLK_E, H), lambda i: (1, i, 0)),
            pl.BlockSpec((BLK_E, E_DIM), lambda i: (i, 0)),
            pl.BlockSpec((OUT_DIM, 2 * H + E_DIM), lambda i: (0, 0)),
            pl.BlockSpec((1, OUT_DIM), lambda i: (0, 0)),
            pl.BlockSpec((8, OUT_DIM), lambda i: (0, 0)),
            pl.BlockSpec((1, OUT_DIM), lambda i: (0, 0)),
            pl.BlockSpec((1, OUT_DIM), lambda i: (0, 0)),
        ],
        out_specs=pl.BlockSpec((BLK_E, H), lambda i: (i, 0)),
        out_shape=jax.ShapeDtypeStruct((N_EDGES, H), jnp.float32),
    )(zcat, zcat, ef, w, bvec, stats, gamma, beta)


N_NODES_PAD = 10240
ROWS_PER_TILE = N_NODES_PAD // NS


def _scatter_body(msg_hbm, eidx_hbm, parts_hbm, sidx, mbuf, acc, sem):
    cid = lax.axis_index("c")
    sid = lax.axis_index("s")
    wid = sid * NC + cid
    base = wid * EDGES_PER_TILE
    stripe = sid * ROWS_PER_TILE

    pltpu.sync_copy(eidx_hbm.at[0, wid], sidx)

    @plsc.parallel_loop(0, CHUNK)
    def _(r):
        for k in range(H // 16):
            mbuf[0, r, pl.ds(k * 16, 16)] = jnp.zeros((16,), jnp.float32)

    for t in range(ROWS_PER_TILE // CHUNK):
        pltpu.sync_copy(mbuf.at[0],
                        acc.at[pl.ds(stripe + t * CHUNK, CHUNK)])
    plsc.subcore_barrier()

    def fire(j, s):
        pltpu.async_copy(msg_hbm.at[pl.ds(base + j * CHUNK, CHUNK)],
                         mbuf.at[s], sem.at[s])

    def process(j, s):
        pltpu.make_async_copy(msg_hbm.at[pl.ds(base + j * CHUNK, CHUNK)],
                              mbuf.at[s], sem.at[s]).wait()
        pltpu.sync_copy(mbuf.at[s], acc.at[sidx.at[j]], add=True)

    fire(0, 0)
    fire(1, 1)

    def chunk(j2, carry):
        for b in range(2):
            j = 2 * j2 + b
            process(j, b)

            @pl.when(j < NCHUNK - 2)
            def _():
                fire(j + 2, b)
        return carry

    lax.fori_loop(0, (NCHUNK - 1) // 2, chunk, 0)
    process(NCHUNK - 1, (NCHUNK - 1) % 2)
    plsc.subcore_barrier()
    pltpu.sync_copy(acc.at[pl.ds(stripe, ROWS_PER_TILE)],
                    parts_hbm.at[cid, pl.ds(stripe, ROWS_PER_TILE)])


def _sc_scatter(msg, eidx4):
    mesh = plsc.VectorSubcoreMesh(core_axis_name="c", subcore_axis_name="s")
    f = pl.kernel(
        _scatter_body,
        out_type=jax.ShapeDtypeStruct((NC, N_NODES_PAD, H), jnp.float32),
        mesh=mesh,
        scratch_types=[
            pltpu.VMEM((NCHUNK, CHUNK), jnp.int32),
            pltpu.VMEM((2, CHUNK, H), jnp.float32),
            pltpu.VMEM_SHARED((N_NODES_PAD, H), jnp.float32),
            pltpu.SemaphoreType.DMA((2,)),
        ],
    )
    return f(msg, eidx4)


def _final_body(nf_ref, p0_ref, p1_ref, out_ref):
    out_ref[...] = nf_ref[...] + p0_ref[0] + p1_ref[0]


def _final_add(nf, parts):
    grid = N_NODES // BLK_N
    return pl.pallas_call(
        _final_body,
        grid=(grid,),
        in_specs=[
            pl.BlockSpec((BLK_N, H), lambda i: (i, 0)),
            pl.BlockSpec((1, BLK_N, H), lambda i: (0, i, 0)),
            pl.BlockSpec((1, BLK_N, H), lambda i: (1, i, 0)),
        ],
        out_specs=pl.BlockSpec((BLK_N, H), lambda i: (i, 0)),
        out_shape=jax.ShapeDtypeStruct((N_NODES, H), jnp.float32),
    )(nf, parts, parts)


@jax.jit
def kernel(node_feats, edge_index, edge_feats, W, b, gamma, beta):
    eidx4 = edge_index.astype(jnp.int32).reshape(2, NW, NCHUNK, CHUNK)
    bvec = b.reshape(1, OUT_DIM)
    gam = gamma.reshape(1, OUT_DIM)
    bet = beta.reshape(1, OUT_DIM)

    p1, p2 = _node_matmul(node_feats, W)
    zcat = _sc_gather(p1, p2, eidx4)
    stats = _stats(zcat, edge_feats, W, bvec)
    msg = _msg(zcat, edge_feats, W, bvec, stats, gam, bet)
    parts = _sc_scatter(msg, eidx4)
    return _final_add(node_feats, parts)
